# baseline (speedup 1.0000x reference)
import functools

import jax
import jax.numpy as jnp
from jax import lax
from jax.experimental import pallas as pl
from jax.experimental.pallas import tpu as pltpu
from jax.experimental.pallas import tpu_sc as plsc

NW = 32
T = 256
IB = 128


def _sc_embed(seq_blk, e1_blk, e2_blk, we, wpe, n, dw, dp):
    d = dw + 2 * dp
    per_w = n // NW
    steps = per_w // T
    k = T // IB

    mesh = plsc.VectorSubcoreMesh(core_axis_name="c", subcore_axis_name="s")

    @functools.partial(
        pl.kernel,
        out_type=jax.ShapeDtypeStruct((n, d), jnp.float32),
        mesh=mesh,
        compiler_params=pltpu.CompilerParams(use_tc_tiling_on_sc=False),
        scratch_types=[
            pltpu.VMEM((2, k, IB), jnp.int32),
            pltpu.VMEM((2, k, IB), jnp.int32),
            pltpu.VMEM((2, k, IB), jnp.int32),
            pltpu.VMEM((2, T, dw), jnp.float32),
            pltpu.VMEM((2, T, dp), jnp.float32),
            pltpu.VMEM((2, T, dp), jnp.float32),
            pltpu.SemaphoreType.DMA,
            pltpu.SemaphoreType.DMA,
            pltpu.SemaphoreType.DMA,
            pltpu.SemaphoreType.DMA,
        ],
    )
    def body(seq_hbm, e1_hbm, e2_hbm, we_hbm, wpe_hbm, out_hbm,
             si_v, p1_v, p2_v, w_v, q1_v, q2_v, gs0, gs1, ws0, ws1):
        cid = lax.axis_index("c")
        sid = lax.axis_index("s")
        wid = sid * 2 + cid
        sblk0 = wid * steps
        gsem = (gs0, gs1)
        wsem = (ws0, ws1)

        def out_copies(b, row0):
            rows = pl.ds(row0, T)
            return [
                pltpu.make_async_copy(
                    w_v.at[b], out_hbm.at[rows, pl.ds(0, dw)], wsem[b]),
                pltpu.make_async_copy(
                    q1_v.at[b], out_hbm.at[rows, pl.ds(dw, dp)], wsem[b]),
                pltpu.make_async_copy(
                    q2_v.at[b], out_hbm.at[rows, pl.ds(dw + dp, dp)],
                    wsem[b]),
            ]

        def gather_copies(b):
            cs = []
            for j in range(k):
                r = pl.ds(j * IB, IB)
                cs.append(pltpu.make_async_copy(
                    we_hbm.at[si_v.at[b].at[j]], w_v.at[b, r], gsem[b]))
                cs.append(pltpu.make_async_copy(
                    wpe_hbm.at[p1_v.at[b].at[j]], q1_v.at[b, r], gsem[b]))
                cs.append(pltpu.make_async_copy(
                    wpe_hbm.at[p2_v.at[b].at[j]], q2_v.at[b, r], gsem[b]))
            return cs

        def step2(t2, carry):
            for b in range(2):
                t = t2 * 2 + b
                row0 = wid * per_w + t * T

                @pl.when(t2 >= 1)
                def _drain_prev():
                    for c in out_copies(b, row0 - 2 * T):
                        c.wait()

                pltpu.sync_copy(seq_hbm.at[sblk0 + t], si_v.at[b])
                pltpu.sync_copy(e1_hbm.at[sblk0 + t], p1_v.at[b])
                pltpu.sync_copy(e2_hbm.at[sblk0 + t], p2_v.at[b])
                gathers = gather_copies(b)
                for c in gathers:
                    c.start()
                for c in gathers:
                    c.wait()
                for c in out_copies(b, row0):
                    c.start()
            return carry

        lax.fori_loop(0, steps // 2, step2, 0)

        last0 = wid * per_w + (steps - 2) * T
        for c in out_copies(0, last0):
            c.wait()
        for c in out_copies(1, last0 + T):
            c.wait()

    return body(seq_blk, e1_blk, e2_blk, we, wpe)


def kernel(seq_inputs, e1_pos_inputs, e2_pos_inputs, we, wpe):
    b, l = seq_inputs.shape
    dw = we.shape[1]
    dp = wpe.shape[1]
    n = b * l
    assert n % (NW * T * 2) == 0 and T % IB == 0
    k = T // IB
    seq_blk = seq_inputs.reshape(n // T, k, IB)
    e1_blk = e1_pos_inputs.reshape(n // T, k, IB)
    e2_blk = e2_pos_inputs.reshape(n // T, k, IB)
    out = _sc_embed(seq_blk, e1_blk, e2_blk, we, wpe, n, dw, dp)
    return out.reshape(b, l, dw + 2 * dp)
